# baseline (speedup 1.0000x reference)
import functools

import jax
import jax.numpy as jnp
from jax import lax
from jax.experimental import pallas as pl
from jax.experimental.pallas import tpu as pltpu
from jax.experimental.pallas import tpu_sc as plsc

N = 10000
E = 320000
NROWS = 10240
NSUB = 16
NCORE = 2
NW = NCORE * NSUB
CHUNK = 64
NCHUNK = 160
NBUF = 2
NGRP = NCHUNK // NBUF
EW = CHUNK * NCHUNK
EPAD = NW * EW
SUB_ROWS = NROWS // NSUB


def _make_sc_agg(C):
    mesh = plsc.VectorSubcoreMesh(core_axis_name="c", subcore_axis_name="s")

    @functools.partial(
        pl.kernel,
        mesh=mesh,
        out_type=jax.ShapeDtypeStruct((NCORE * NROWS, C), jnp.float32),
        scratch_types=[
            [pltpu.VMEM((NBUF, 2, CHUNK), jnp.int32) for _ in range(2)],
            [pltpu.VMEM((CHUNK, C), jnp.float32) for _ in range(2 * NBUF)],
            pltpu.VMEM_SHARED((NROWS, C), jnp.float32),
            [pltpu.SemaphoreType.DMA for _ in range(2 * NBUF)],
            [pltpu.SemaphoreType.DMA for _ in range(2 * NBUF)],
            [pltpu.SemaphoreType.DMA for _ in range(2)],
        ],
    )
    def sc_agg(x_hbm, idx_hbm, zero_hbm, out_hbm, ibuf, rows, acc, gsem, ssem,
               isem):
        c = lax.axis_index("c")
        s = lax.axis_index("s")
        wid = s * jnp.int32(NCORE) + c
        r0 = s * jnp.int32(SUB_ROWS)
        ibase = wid * jnp.int32(NCHUNK)

        def idx_src(g):
            return idx_hbm.at[pl.ds(ibase + g * jnp.int32(NBUF), NBUF)]

        def start_idx(g, p):
            pltpu.async_copy(idx_src(g), ibuf[p], isem[p])

        def wait_idx(g, p):
            pltpu.make_async_copy(idx_src(g), ibuf[p], isem[p]).wait()

        def start_gather(p, b):
            pltpu.async_copy(
                x_hbm.at[ibuf[p].at[jnp.int32(b), jnp.int32(0)]],
                rows[2 * p + b], gsem[2 * p + b])

        def wait_gather(p, b):
            pltpu.make_async_copy(
                x_hbm.at[ibuf[p].at[jnp.int32(b), jnp.int32(0)]],
                rows[2 * p + b], gsem[2 * p + b]).wait()

        def start_scatter(p, b):
            return pltpu.async_copy(
                rows[2 * p + b], acc.at[ibuf[p].at[jnp.int32(b), jnp.int32(1)]],
                ssem[2 * p + b], add=True)

        zcp = pltpu.async_copy(zero_hbm.at[pl.ds(r0, SUB_ROWS)],
                               acc.at[pl.ds(r0, SUB_ROWS)], gsem[0])
        pltpu.sync_copy(idx_src(jnp.int32(0)), ibuf[0])
        start_idx(jnp.int32(1), 1)
        zcp.wait()
        plsc.subcore_barrier()
        for b in range(NBUF):
            start_gather(0, b)

        def do_group(g, p, last):
            scats = []
            for b in range(NBUF):
                wait_gather(p, b)
                scats.append(start_scatter(p, b))
            if not last:
                wait_idx(g + 1, 1 - p)
                for b in range(NBUF):
                    start_gather(1 - p, b)
            for b in range(NBUF):
                scats[b].wait()

        def body(i, carry):
            g = i * jnp.int32(2)
            do_group(g, 0, False)
            start_idx(g + jnp.int32(2), 0)
            do_group(g + jnp.int32(1), 1, False)
            start_idx(g + jnp.int32(3), 1)
            return carry

        lax.fori_loop(jnp.int32(0), jnp.int32(NGRP // 2 - 1), body, jnp.int32(0),
                      unroll=False)
        do_group(jnp.int32(NGRP - 2), 0, False)
        do_group(jnp.int32(NGRP - 1), 1, True)

        plsc.subcore_barrier()
        pltpu.sync_copy(
            acc.at[pl.ds(r0, SUB_ROWS)],
            out_hbm.at[pl.ds(c * jnp.int32(NROWS) + r0, SUB_ROWS)],
        )

    return sc_agg


_sc_agg_128 = _make_sc_agg(128)


def _make_tc_transform(delta, relu, first_prec=lax.Precision.HIGHEST):
    def body(x_ref, p_ref, d_ref, w1_ref, b1_ref, g1_ref, be1_ref,
             w2_ref, b2_ref, g2_ref, be2_ref, o_ref):
        p = p_ref[:N, :] + p_ref[NROWS:NROWS + N, :]
        x = x_ref[...]
        invd = 1.0 / d_ref[...]
        if delta:
            out = (p - p[:, 3:4] * x) * invd
        else:
            out = x + p * invd
        y = lax.dot_general(out, w1_ref[...], (((1,), (1,)), ((), ())),
                            preferred_element_type=jnp.float32,
                            precision=first_prec) + b1_ref[...]
        mu = jnp.mean(y, axis=0, keepdims=True)
        var = jnp.mean((y - mu) ** 2, axis=0, keepdims=True)
        y = (y - mu) * lax.rsqrt(var + 1e-5) * g1_ref[...] + be1_ref[...]
        z = lax.dot_general(y, w2_ref[...], (((1,), (1,)), ((), ())),
                            preferred_element_type=jnp.float32,
                            precision=lax.Precision.HIGHEST) + b2_ref[...]
        mu = jnp.mean(z, axis=0, keepdims=True)
        var = jnp.mean((z - mu) ** 2, axis=0, keepdims=True)
        z = (z - mu) * lax.rsqrt(var + 1e-5) * g2_ref[...] + be2_ref[...]
        if relu:
            z = jnp.maximum(z, 0.0)
        o_ref[...] = z

    return pl.pallas_call(
        body, out_shape=jax.ShapeDtypeStruct((N, 128), jnp.float32))


_tc_layer0 = _make_tc_transform(delta=True, relu=False,
                                first_prec=lax.Precision.DEFAULT)
_tc_layer1 = _make_tc_transform(delta=False, relu=True)
_tc_layer2 = _make_tc_transform(delta=False, relu=False)


def kernel(vertices, edges, dofs, W0a, b0a, Wm, bm, gamma, beta):
    v = vertices[0].astype(jnp.float32)
    fr = edges[0, 0, 0].astype(jnp.int32)
    to = edges[0, 0, 1].astype(jnp.int32)
    dcol = dofs[0, 0].astype(jnp.float32)[:, None]

    fr_p = jnp.concatenate([fr, jnp.full((EPAD - E,), N, jnp.int32)]
                           ).reshape(NW * NCHUNK, 1, CHUNK)
    to_p = jnp.concatenate([to, jnp.zeros((EPAD - E,), jnp.int32)]
                           ).reshape(NW * NCHUNK, 1, CHUNK)
    idx = jnp.concatenate([to_p, fr_p], axis=1)

    z128 = jnp.zeros((NROWS, 128), jnp.float32)

    def row(a):
        return a.astype(jnp.float32)[None, :]

    v128 = jnp.concatenate(
        [v, jnp.ones((N, 1), jnp.float32), jnp.zeros((N, 124), jnp.float32)], 1)
    W0a_pad = jnp.concatenate([W0a.astype(jnp.float32),
                               jnp.zeros((128, 125), jnp.float32)], 1)
    parts = _sc_agg_128(v128, idx, z128)
    x = _tc_layer0(v128, parts, dcol, W0a_pad, row(b0a), row(gamma[0]), row(beta[0]),
                   Wm[0].astype(jnp.float32), row(bm[0]), row(gamma[1]), row(beta[1]))

    parts = _sc_agg_128(x, idx, z128)
    x = _tc_layer1(x, parts, dcol, Wm[1].astype(jnp.float32), row(bm[1]),
                   row(gamma[2]), row(beta[2]), Wm[2].astype(jnp.float32),
                   row(bm[2]), row(gamma[3]), row(beta[3]))

    parts = _sc_agg_128(x, idx, z128)
    x = _tc_layer2(x, parts, dcol, Wm[3].astype(jnp.float32), row(bm[3]),
                   row(gamma[4]), row(beta[4]), Wm[4].astype(jnp.float32),
                   row(bm[4]), row(gamma[5]), row(beta[5]))

    return jnp.transpose(x)[None].astype(jnp.float64)


# speedup vs baseline: 1.2825x; 1.2825x over previous
import functools

import jax
import jax.numpy as jnp
from jax import lax
from jax.experimental import pallas as pl
from jax.experimental.pallas import tpu as pltpu
from jax.experimental.pallas import tpu_sc as plsc

N = 10000
E = 320000
NROWS = 10240
NSUB = 16
NCORE = 2
NW = NCORE * NSUB
CHUNK = 128
NC0 = 126
NC1 = 34
TCH = NSUB * (NC0 + NC1)
EPAD = TCH * CHUNK
SUB_ROWS = NROWS // NSUB


def _make_sc_agg(C):
    mesh = plsc.VectorSubcoreMesh(core_axis_name="c", subcore_axis_name="s")

    @functools.partial(
        pl.kernel,
        mesh=mesh,
        out_type=jax.ShapeDtypeStruct((NCORE * NROWS, C), jnp.float32),
        scratch_types=[
            [pltpu.VMEM((1, 2, CHUNK), jnp.int32) for _ in range(2)],
            [pltpu.VMEM((CHUNK, C), jnp.float32) for _ in range(2)],
            pltpu.VMEM_SHARED((NROWS, C), jnp.float32),
            [pltpu.SemaphoreType.DMA for _ in range(2)],
            [pltpu.SemaphoreType.DMA for _ in range(2)],
            [pltpu.SemaphoreType.DMA for _ in range(2)],
        ],
    )
    def sc_agg(x_hbm, idx_hbm, zero_hbm, out_hbm, ibuf, rows, acc, gsem, ssem,
               isem):
        c = lax.axis_index("c")
        s = lax.axis_index("s")
        r0 = s * jnp.int32(SUB_ROWS)

        def pipeline(nchunk, ibase):
            def idx_src(g):
                return idx_hbm.at[pl.ds(ibase + g, 1)]

            def start_idx(g, p):
                pltpu.async_copy(idx_src(g), ibuf[p], isem[p])

            def wait_idx(g, p):
                pltpu.make_async_copy(idx_src(g), ibuf[p], isem[p]).wait()

            def start_gather(p):
                pltpu.async_copy(
                    x_hbm.at[ibuf[p].at[jnp.int32(0), jnp.int32(0)]],
                    rows[p], gsem[p])

            def wait_gather(p):
                pltpu.make_async_copy(
                    x_hbm.at[ibuf[p].at[jnp.int32(0), jnp.int32(0)]],
                    rows[p], gsem[p]).wait()

            def start_scatter(p):
                return pltpu.async_copy(
                    rows[p], acc.at[ibuf[p].at[jnp.int32(0), jnp.int32(1)]],
                    ssem[p], add=True)

            pltpu.sync_copy(idx_src(jnp.int32(0)), ibuf[0])
            start_idx(jnp.int32(1), 1)
            start_gather(0)

            def do_group(g, p, last):
                wait_gather(p)
                scat = start_scatter(p)
                if not last:
                    wait_idx(g + 1, 1 - p)
                    start_gather(1 - p)
                scat.wait()

            def body(i, carry):
                g = i * jnp.int32(2)
                do_group(g, 0, False)
                start_idx(g + jnp.int32(2), 0)
                do_group(g + jnp.int32(1), 1, False)
                start_idx(g + jnp.int32(3), 1)
                return carry

            lax.fori_loop(jnp.int32(0), jnp.int32(nchunk // 2 - 1), body,
                          jnp.int32(0), unroll=False)
            do_group(jnp.int32(nchunk - 2), 0, False)
            do_group(jnp.int32(nchunk - 1), 1, True)

        pltpu.sync_copy(zero_hbm.at[pl.ds(r0, SUB_ROWS)],
                        acc.at[pl.ds(r0, SUB_ROWS)])
        plsc.subcore_barrier()

        @pl.when(c == 0)
        def _():
            pipeline(NC0, s * jnp.int32(NC0))

        @pl.when(c == 1)
        def _():
            pipeline(NC1, jnp.int32(NSUB * NC0) + s * jnp.int32(NC1))

        plsc.subcore_barrier()
        pltpu.sync_copy(
            acc.at[pl.ds(r0, SUB_ROWS)],
            out_hbm.at[pl.ds(c * jnp.int32(NROWS) + r0, SUB_ROWS)],
        )

    return sc_agg


_sc_agg_128 = _make_sc_agg(128)


def _make_tc_transform(delta, relu, first_prec=lax.Precision.HIGHEST):
    def body(x_ref, p_ref, d_ref, w1_ref, b1_ref, g1_ref, be1_ref,
             w2_ref, b2_ref, g2_ref, be2_ref, o_ref):
        p = p_ref[:N, :] + p_ref[NROWS:NROWS + N, :]
        x = x_ref[...]
        invd = 1.0 / d_ref[...]
        if delta:
            out = (p - p[:, 3:4] * x) * invd
        else:
            out = x + p * invd
        y = lax.dot_general(out, w1_ref[...], (((1,), (1,)), ((), ())),
                            preferred_element_type=jnp.float32,
                            precision=first_prec) + b1_ref[...]
        mu = jnp.mean(y, axis=0, keepdims=True)
        var = jnp.mean((y - mu) ** 2, axis=0, keepdims=True)
        y = (y - mu) * lax.rsqrt(var + 1e-5) * g1_ref[...] + be1_ref[...]
        z = lax.dot_general(y, w2_ref[...], (((1,), (1,)), ((), ())),
                            preferred_element_type=jnp.float32,
                            precision=lax.Precision.HIGHEST) + b2_ref[...]
        mu = jnp.mean(z, axis=0, keepdims=True)
        var = jnp.mean((z - mu) ** 2, axis=0, keepdims=True)
        z = (z - mu) * lax.rsqrt(var + 1e-5) * g2_ref[...] + be2_ref[...]
        if relu:
            z = jnp.maximum(z, 0.0)
        o_ref[...] = z

    return pl.pallas_call(
        body, out_shape=jax.ShapeDtypeStruct((N, 128), jnp.float32))


_tc_layer0 = _make_tc_transform(delta=True, relu=False,
                                first_prec=lax.Precision.DEFAULT)
_tc_layer1 = _make_tc_transform(delta=False, relu=True)
_tc_layer2 = _make_tc_transform(delta=False, relu=False)


def kernel(vertices, edges, dofs, W0a, b0a, Wm, bm, gamma, beta):
    v = vertices[0].astype(jnp.float32)
    fr = edges[0, 0, 0].astype(jnp.int32)
    to = edges[0, 0, 1].astype(jnp.int32)
    dcol = dofs[0, 0].astype(jnp.float32)[:, None]

    fr_p = jnp.concatenate([fr, jnp.full((EPAD - E,), N, jnp.int32)]
                           ).reshape(TCH, 1, CHUNK)
    to_p = jnp.concatenate([to, jnp.zeros((EPAD - E,), jnp.int32)]
                           ).reshape(TCH, 1, CHUNK)
    idx = jnp.concatenate([to_p, fr_p], axis=1)

    z128 = jnp.zeros((NROWS, 128), jnp.float32)

    def row(a):
        return a.astype(jnp.float32)[None, :]

    v128 = jnp.concatenate(
        [v, jnp.ones((N, 1), jnp.float32), jnp.zeros((N, 124), jnp.float32)], 1)
    W0a_pad = jnp.concatenate([W0a.astype(jnp.float32),
                               jnp.zeros((128, 125), jnp.float32)], 1)
    parts = _sc_agg_128(v128, idx, z128)
    x = _tc_layer0(v128, parts, dcol, W0a_pad, row(b0a), row(gamma[0]), row(beta[0]),
                   Wm[0].astype(jnp.float32), row(bm[0]), row(gamma[1]), row(beta[1]))

    parts = _sc_agg_128(x, idx, z128)
    x = _tc_layer1(x, parts, dcol, Wm[1].astype(jnp.float32), row(bm[1]),
                   row(gamma[2]), row(beta[2]), Wm[2].astype(jnp.float32),
                   row(bm[2]), row(gamma[3]), row(beta[3]))

    parts = _sc_agg_128(x, idx, z128)
    x = _tc_layer2(x, parts, dcol, Wm[3].astype(jnp.float32), row(bm[3]),
                   row(gamma[4]), row(beta[4]), Wm[4].astype(jnp.float32),
                   row(bm[4]), row(gamma[5]), row(beta[5]))

    return jnp.transpose(x)[None].astype(jnp.float64)


# speedup vs baseline: 1.3042x; 1.0169x over previous
import functools

import jax
import jax.numpy as jnp
from jax import lax
from jax.experimental import pallas as pl
from jax.experimental.pallas import tpu as pltpu
from jax.experimental.pallas import tpu_sc as plsc

N = 10000
E = 320000
NROWS = 10240
NSUB = 16
NCORE = 2
NW = NCORE * NSUB
CHUNK = 128
NC0 = 138
NC1 = 22
TCH = NSUB * (NC0 + NC1)
EPAD = TCH * CHUNK
SUB_ROWS = NROWS // NSUB


def _make_sc_agg(C):
    mesh = plsc.VectorSubcoreMesh(core_axis_name="c", subcore_axis_name="s")

    @functools.partial(
        pl.kernel,
        mesh=mesh,
        out_type=jax.ShapeDtypeStruct((NCORE * NROWS, C), jnp.float32),
        scratch_types=[
            [pltpu.VMEM((1, 2, CHUNK), jnp.int32) for _ in range(2)],
            [pltpu.VMEM((CHUNK, C), jnp.float32) for _ in range(2)],
            pltpu.VMEM_SHARED((NROWS, C), jnp.float32),
            [pltpu.SemaphoreType.DMA for _ in range(2)],
            [pltpu.SemaphoreType.DMA for _ in range(2)],
            [pltpu.SemaphoreType.DMA for _ in range(2)],
        ],
    )
    def sc_agg(x_hbm, idx_hbm, zero_hbm, out_hbm, ibuf, rows, acc, gsem, ssem,
               isem):
        c = lax.axis_index("c")
        s = lax.axis_index("s")
        r0 = s * jnp.int32(SUB_ROWS)

        def pipeline(nchunk, ibase):
            def idx_src(g):
                return idx_hbm.at[pl.ds(ibase + g, 1)]

            def start_idx(g, p):
                pltpu.async_copy(idx_src(g), ibuf[p], isem[p])

            def wait_idx(g, p):
                pltpu.make_async_copy(idx_src(g), ibuf[p], isem[p]).wait()

            def start_gather(p):
                pltpu.async_copy(
                    x_hbm.at[ibuf[p].at[jnp.int32(0), jnp.int32(0)]],
                    rows[p], gsem[p])

            def wait_gather(p):
                pltpu.make_async_copy(
                    x_hbm.at[ibuf[p].at[jnp.int32(0), jnp.int32(0)]],
                    rows[p], gsem[p]).wait()

            def start_scatter(p):
                return pltpu.async_copy(
                    rows[p], acc.at[ibuf[p].at[jnp.int32(0), jnp.int32(1)]],
                    ssem[p], add=True)

            pltpu.sync_copy(idx_src(jnp.int32(0)), ibuf[0])
            start_idx(jnp.int32(1), 1)
            start_gather(0)

            def do_group(g, p, last):
                wait_gather(p)
                scat = start_scatter(p)
                if not last:
                    wait_idx(g + 1, 1 - p)
                    start_gather(1 - p)
                scat.wait()

            def body(i, carry):
                g = i * jnp.int32(2)
                do_group(g, 0, False)
                start_idx(g + jnp.int32(2), 0)
                do_group(g + jnp.int32(1), 1, False)
                start_idx(g + jnp.int32(3), 1)
                return carry

            lax.fori_loop(jnp.int32(0), jnp.int32(nchunk // 2 - 1), body,
                          jnp.int32(0), unroll=False)
            do_group(jnp.int32(nchunk - 2), 0, False)
            do_group(jnp.int32(nchunk - 1), 1, True)

        pltpu.sync_copy(zero_hbm.at[pl.ds(r0, SUB_ROWS)],
                        acc.at[pl.ds(r0, SUB_ROWS)])
        plsc.subcore_barrier()

        @pl.when(c == 0)
        def _():
            pipeline(NC0, s * jnp.int32(NC0))

        @pl.when(c == 1)
        def _():
            pipeline(NC1, jnp.int32(NSUB * NC0) + s * jnp.int32(NC1))

        plsc.subcore_barrier()
        pltpu.sync_copy(
            acc.at[pl.ds(r0, SUB_ROWS)],
            out_hbm.at[pl.ds(c * jnp.int32(NROWS) + r0, SUB_ROWS)],
        )

    return sc_agg


_sc_agg_128 = _make_sc_agg(128)


def _make_tc_transform(delta, relu, first_prec=lax.Precision.HIGHEST):
    def body(x_ref, p_ref, d_ref, w1_ref, b1_ref, g1_ref, be1_ref,
             w2_ref, b2_ref, g2_ref, be2_ref, o_ref):
        p = p_ref[:N, :] + p_ref[NROWS:NROWS + N, :]
        x = x_ref[...]
        invd = 1.0 / d_ref[...]
        if delta:
            out = (p - p[:, 3:4] * x) * invd
        else:
            out = x + p * invd
        y = lax.dot_general(out, w1_ref[...], (((1,), (1,)), ((), ())),
                            preferred_element_type=jnp.float32,
                            precision=first_prec) + b1_ref[...]
        mu = jnp.mean(y, axis=0, keepdims=True)
        var = jnp.mean((y - mu) ** 2, axis=0, keepdims=True)
        y = (y - mu) * lax.rsqrt(var + 1e-5) * g1_ref[...] + be1_ref[...]
        z = lax.dot_general(y, w2_ref[...], (((1,), (1,)), ((), ())),
                            preferred_element_type=jnp.float32,
                            precision=lax.Precision.HIGHEST) + b2_ref[...]
        mu = jnp.mean(z, axis=0, keepdims=True)
        var = jnp.mean((z - mu) ** 2, axis=0, keepdims=True)
        z = (z - mu) * lax.rsqrt(var + 1e-5) * g2_ref[...] + be2_ref[...]
        if relu:
            z = jnp.maximum(z, 0.0)
        o_ref[...] = z

    return pl.pallas_call(
        body, out_shape=jax.ShapeDtypeStruct((N, 128), jnp.float32))


_tc_layer0 = _make_tc_transform(delta=True, relu=False,
                                first_prec=lax.Precision.DEFAULT)
_tc_layer1 = _make_tc_transform(delta=False, relu=True)
_tc_layer2 = _make_tc_transform(delta=False, relu=False)


def kernel(vertices, edges, dofs, W0a, b0a, Wm, bm, gamma, beta):
    v = vertices[0].astype(jnp.float32)
    fr = edges[0, 0, 0].astype(jnp.int32)
    to = edges[0, 0, 1].astype(jnp.int32)
    dcol = dofs[0, 0].astype(jnp.float32)[:, None]

    fr_p = jnp.concatenate([fr, jnp.full((EPAD - E,), N, jnp.int32)]
                           ).reshape(TCH, 1, CHUNK)
    to_p = jnp.concatenate([to, jnp.zeros((EPAD - E,), jnp.int32)]
                           ).reshape(TCH, 1, CHUNK)
    idx = jnp.concatenate([to_p, fr_p], axis=1)

    z128 = jnp.zeros((NROWS, 128), jnp.float32)

    def row(a):
        return a.astype(jnp.float32)[None, :]

    v128 = jnp.concatenate(
        [v, jnp.ones((N, 1), jnp.float32), jnp.zeros((N, 124), jnp.float32)], 1)
    W0a_pad = jnp.concatenate([W0a.astype(jnp.float32),
                               jnp.zeros((128, 125), jnp.float32)], 1)
    parts = _sc_agg_128(v128, idx, z128)
    x = _tc_layer0(v128, parts, dcol, W0a_pad, row(b0a), row(gamma[0]), row(beta[0]),
                   Wm[0].astype(jnp.float32), row(bm[0]), row(gamma[1]), row(beta[1]))

    parts = _sc_agg_128(x, idx, z128)
    x = _tc_layer1(x, parts, dcol, Wm[1].astype(jnp.float32), row(bm[1]),
                   row(gamma[2]), row(beta[2]), Wm[2].astype(jnp.float32),
                   row(bm[2]), row(gamma[3]), row(beta[3]))

    parts = _sc_agg_128(x, idx, z128)
    x = _tc_layer2(x, parts, dcol, Wm[3].astype(jnp.float32), row(bm[3]),
                   row(gamma[4]), row(beta[4]), Wm[4].astype(jnp.float32),
                   row(bm[4]), row(gamma[5]), row(beta[5]))

    return jnp.transpose(x)[None].astype(jnp.float64)
